# baseline (speedup 1.0000x reference)
import jax
import jax.numpy as jnp
from jax import lax
from jax.experimental import pallas as pl
from jax.experimental.pallas import tpu as pltpu
from jax.experimental.pallas import tpu_sc as plsc

_EPS = 0.01
_N = 4096
_NW = 32
_MAPS_PER_W = _N // _NW
_MBLK = 4
_TBLK = _MBLK * 8
_NBLK = _MAPS_PER_W // _MBLK
_NT = (_N + 1) * 8

_mesh = plsc.VectorSubcoreMesh(core_axis_name="c", subcore_axis_name="s")


def _toz_sc_body(x_hbm, z_hbm, e_hbm, out_hbm, buf0, buf1, xbuf, sem0, sem1,
                 semo0, semo1):
    wid = lax.axis_index("s") * 2 + lax.axis_index("c")

    @pl.when(wid == 0)
    def _():
        pltpu.sync_copy(x_hbm, xbuf)
        pltpu.sync_copy(xbuf, out_hbm.at[pl.ds(0, 8)])

    pltpu.sync_copy(z_hbm, buf0)
    pltpu.sync_copy(z_hbm, buf1)

    bufs = (buf0, buf1)
    sems = (sem0, sem1)
    osems = (semo0, semo1)
    copies = [None, None]
    prev_tt = [[None] * _MBLK, [None] * _MBLK]

    j_base = 1 + wid * _MAPS_PER_W
    for t in range(_NBLK):
        b = t % 2
        buf = bufs[b]
        if copies[b] is not None:
            copies[b].wait()
        pokes = []
        for jj in range(_MBLK):
            j = j_base + t * _MBLK + jj
            fcode = j - 1
            r = lax.shift_right_logical(fcode, 6)
            c = lax.bitwise_and(fcode, 63)
            tr = lax.shift_right_logical(r, 3)
            tt = jj * 8 + tr
            k = lax.bitwise_and(r, 7) * 64 + c
            old = prev_tt[b][jj]
            if old is not None:
                @pl.when(old != tt)
                def _(old=old):
                    pltpu.sync_copy(z_hbm.at[pl.ds(0, 1)],
                                    buf.at[pl.ds(old, 1)])
            pokes.append(pltpu.async_copy(e_hbm.at[pl.ds(k, 1)],
                                          buf.at[pl.ds(tt, 1)], osems[b]))
            prev_tt[b][jj] = tt
        for p in pokes:
            p.wait()
        t0 = 8 * (j_base + t * _MBLK)
        copies[b] = pltpu.async_copy(buf, out_hbm.at[pl.ds(t0, _TBLK)],
                                     sems[b])

    copies[0].wait()
    copies[1].wait()


def kernel(x):
    xt = x.reshape(8, 8, 64)
    zeros = jnp.zeros((_TBLK, 8, 64), jnp.float32)
    etab = (_EPS * jnp.eye(512, dtype=jnp.float32)).reshape(512, 8, 64)
    out = pl.kernel(
        _toz_sc_body,
        out_type=jax.ShapeDtypeStruct((_NT, 8, 64), jnp.float32),
        mesh=_mesh,
        scratch_types=[
            pltpu.VMEM((_TBLK, 8, 64), jnp.float32),
            pltpu.VMEM((_TBLK, 8, 64), jnp.float32),
            pltpu.VMEM((8, 8, 64), jnp.float32),
            pltpu.SemaphoreType.DMA,
            pltpu.SemaphoreType.DMA,
            pltpu.SemaphoreType.DMA,
            pltpu.SemaphoreType.DMA,
        ],
    )(xt, zeros, etab)
    return out.reshape(_N + 1, 1, 64, 64)


# speedup vs baseline: 1.8284x; 1.8284x over previous
import jax
import jax.numpy as jnp
from jax.experimental import pallas as pl
from jax.experimental.pallas import tpu as pltpu

_EPS = 0.01
_N = 4096
_BLK = 256


def _toz_body(x_ref, o_ref):
    i = pl.program_id(0)
    row = i * _BLK + jax.lax.broadcasted_iota(jnp.int32, (_BLK, _N), 0)
    col = jax.lax.broadcasted_iota(jnp.int32, (_BLK, _N), 1)
    diag = jnp.where(row - 1 == col, _EPS, 0.0).astype(jnp.float32)
    o_ref[...] = jnp.where(row == 0, x_ref[...], diag)


def kernel(x):
    xf = x.reshape(1, _N)
    grid = (_N + 1 + _BLK - 1) // _BLK
    out = pl.pallas_call(
        _toz_body,
        grid=(grid,),
        in_specs=[pl.BlockSpec((1, _N), lambda i: (0, 0))],
        out_specs=pl.BlockSpec((_BLK, _N), lambda i: (i, 0)),
        out_shape=jax.ShapeDtypeStruct((_N + 1, _N), jnp.float32),
    )(xf)
    return out.reshape(_N + 1, 1, 64, 64)
